# baseline (speedup 1.0000x reference)
import functools

import jax
import jax.numpy as jnp
from jax import lax
from jax.experimental import pallas as pl
from jax.experimental.pallas import tpu as pltpu
from jax.experimental.pallas import tpu_sc as plsc

_COMMITMENT_COST = 0.25


def _dist_argmin_body(z_ref, cb_ref, cb2_ref, idx_ref, mind_ref):
    zb = z_ref[0]
    cb = cb_ref[...]
    cb2 = cb2_ref[...]
    mm2 = lax.dot_general(cb2, zb, (((1,), (0,)), ((), ())),
                          preferred_element_type=jnp.float32)
    cn = jnp.sum(cb * cb, axis=1, keepdims=True)
    zn = jnp.sum(zb * zb, axis=0, keepdims=True)
    d = (zn + cn) + mm2
    m = jnp.min(d, axis=0, keepdims=True)
    k, p = d.shape
    row = lax.broadcasted_iota(jnp.int32, (k, p), 0)
    idx = jnp.min(jnp.where(d == m, row, jnp.int32(k)), axis=0)
    idx_ref[0, 0, :] = idx
    mind_ref[0, 0, :] = m[0]


def _dist_argmin(zr, codebook):
    b, dim, p = zr.shape
    k = codebook.shape[0]
    return pl.pallas_call(
        _dist_argmin_body,
        grid=(b,),
        in_specs=[
            pl.BlockSpec((1, dim, p), lambda i: (i, 0, 0)),
            pl.BlockSpec((k, dim), lambda i: (0, 0)),
            pl.BlockSpec((k, dim), lambda i: (0, 0)),
        ],
        out_specs=[
            pl.BlockSpec((1, 1, p), lambda i: (i, 0, 0)),
            pl.BlockSpec((1, 1, p), lambda i: (i, 0, 0)),
        ],
        out_shape=[
            jax.ShapeDtypeStruct((b, 1, p), jnp.int32),
            jax.ShapeDtypeStruct((b, 1, p), jnp.float32),
        ],
    )(zr, codebook, jnp.float32(-2.0) * codebook)


def _make_sc_gather(n, dim, nc, nw, chunks, chunk):
    b_per_w = chunks * chunk
    mesh = plsc.VectorSubcoreMesh(core_axis_name="c", subcore_axis_name="s")

    @functools.partial(
        pl.kernel, mesh=mesh,
        compiler_params=pltpu.CompilerParams(use_tc_tiling_on_sc=False),
        out_type=jax.ShapeDtypeStruct((n, dim), jnp.float32),
        scratch_types=[
            pltpu.VMEM((chunks, chunk), jnp.int32),
            pltpu.VMEM((b_per_w, dim), jnp.float32),
            pltpu.SemaphoreType.DMA,
        ],
    )
    def gather_kernel(table_hbm, idx_hbm, out_hbm, idx_v, rows_v, sem):
        wid = lax.axis_index("s") * nc + lax.axis_index("c")
        pltpu.sync_copy(idx_hbm.at[wid], idx_v)
        copies = []
        for j in range(chunks):
            copies.append(pltpu.async_copy(
                table_hbm.at[idx_v.at[j]],
                rows_v.at[pl.ds(j * chunk, chunk)], sem))
        for c in copies:
            c.wait()
        pltpu.sync_copy(rows_v, out_hbm.at[pl.ds(wid * b_per_w, b_per_w)])

    return gather_kernel


def kernel(z, codebook):
    b, dim, h, w = z.shape
    p = h * w
    n = b * p
    zr = z.reshape(b, dim, p)
    idx3, mind = _dist_argmin(zr, codebook)
    idx_flat = idx3.reshape(n)

    mse = jnp.sum(mind) / (n * dim)
    loss = mse + _COMMITMENT_COST * mse

    info = plsc.get_sparse_core_info()
    nc, ns = info.num_cores, info.num_subcores
    nw = nc * ns
    chunk = 128
    chunks = n // (nw * chunk)
    gather_fn = _make_sc_gather(n, dim, nc, nw, chunks, chunk)
    zq_flat = gather_fn(codebook, idx_flat.reshape(nw, chunks, chunk))

    z_q = zq_flat.reshape(b, h, w, dim).transpose(0, 3, 1, 2)
    return z_q, loss, idx_flat


# speedup vs baseline: 1.2922x; 1.2922x over previous
import jax
import jax.numpy as jnp
from jax import lax
from jax.experimental import pallas as pl

_COMMITMENT_COST = 0.25


def _vq_body(z_ref, cb_ref, cb2_ref, zq_ref, idx_ref, mind_ref):
    zb = z_ref[0]
    cb = cb_ref[...]
    cb2 = cb2_ref[...]
    mm2 = lax.dot_general(cb2, zb, (((1,), (0,)), ((), ())),
                          preferred_element_type=jnp.float32)
    cn = jnp.sum(cb * cb, axis=1, keepdims=True)
    zn = jnp.sum(zb * zb, axis=0, keepdims=True)
    d = (zn + cn) + mm2
    m = jnp.min(d, axis=0, keepdims=True)
    k, p = d.shape
    row = lax.broadcasted_iota(jnp.int32, (k, p), 0)
    idx = jnp.min(jnp.where(d == m, row, jnp.int32(k)), axis=0)
    onehot = jnp.where(row == idx[None, :], jnp.float32(1.0), jnp.float32(0.0))
    zqt = lax.dot_general(cb, onehot, (((0,), (0,)), ((), ())),
                          preferred_element_type=jnp.float32)
    zq_ref[0] = zqt
    idx_ref[0, 0, :] = idx
    mind_ref[0, 0, :] = m[0]


def kernel(z, codebook):
    b, dim, h, w = z.shape
    p = h * w
    n = b * p
    k = codebook.shape[0]
    zr = z.reshape(b, dim, p)
    zqt, idx3, mind = pl.pallas_call(
        _vq_body,
        grid=(b,),
        in_specs=[
            pl.BlockSpec((1, dim, p), lambda i: (i, 0, 0)),
            pl.BlockSpec((k, dim), lambda i: (0, 0)),
            pl.BlockSpec((k, dim), lambda i: (0, 0)),
        ],
        out_specs=[
            pl.BlockSpec((1, dim, p), lambda i: (i, 0, 0)),
            pl.BlockSpec((1, 1, p), lambda i: (i, 0, 0)),
            pl.BlockSpec((1, 1, p), lambda i: (i, 0, 0)),
        ],
        out_shape=[
            jax.ShapeDtypeStruct((b, dim, p), jnp.float32),
            jax.ShapeDtypeStruct((b, 1, p), jnp.int32),
            jax.ShapeDtypeStruct((b, 1, p), jnp.float32),
        ],
    )(zr, codebook, jnp.float32(-2.0) * codebook)

    mse = jnp.sum(mind) / (n * dim)
    loss = mse + _COMMITMENT_COST * mse
    return zqt.reshape(b, dim, h, w), loss, idx3.reshape(n)
